# baseline (speedup 1.0000x reference)
import functools

import jax
import jax.numpy as jnp
from jax import lax
from jax.experimental import pallas as pl
from jax.experimental.pallas import tpu as pltpu
from jax.experimental.pallas import tpu_sc as plsc

N = 10000
E = 320000
H = 128
G = 64
EPS = 1e-5

NC = 2
NS = 16
NW = NC * NS
KE = 128
NCHE = 80
EPT = NCHE * KE
EPAD = NW * EPT
NPAD = 10240
ZCH = NPAD // NS
NR = 10112
ZR = NR // NS
NCHH = NCHE // 2
_WB = [(0, 128), (128, 128), (256, 128), (384, 128), (512, 120)]

RB = 1000
NBLK = N // RB

_mesh = plsc.VectorSubcoreMesh(core_axis_name="c", subcore_axis_name="s")


@functools.partial(
    pl.kernel,
    mesh=_mesh,
    out_type=jax.ShapeDtypeStruct((NC, NPAD), jnp.float32),
    scratch_types=[
        pltpu.VMEM_SHARED((NPAD,), jnp.float32),
        pltpu.VMEM((NCHE, KE), jnp.int32),
        pltpu.VMEM((NCHE, KE), jnp.float32),
        pltpu.VMEM((ZCH,), jnp.float32),
    ],
)
def _deg_call(col_hbm, ew_hbm, out_hbm, acc, colv, ewv, zbuf):
    c = lax.axis_index("c")
    s = lax.axis_index("s")
    wid = c * NS + s
    zero = jnp.zeros((16,), jnp.float32)

    def zb(i, _):
        zbuf[pl.ds(pl.multiple_of(i * 16, 16), 16)] = zero
        return 0

    lax.fori_loop(0, ZCH // 16, zb, 0)
    pltpu.sync_copy(zbuf, acc.at[pl.ds(s * ZCH, ZCH)])
    pltpu.sync_copy(col_hbm.at[wid], colv)
    pltpu.sync_copy(ew_hbm.at[wid], ewv)
    plsc.subcore_barrier()

    def ch(j, _):
        pltpu.sync_copy(ewv.at[j], acc.at[colv.at[j]], add=True)
        return 0

    lax.fori_loop(0, NCHE, ch, 0)
    plsc.subcore_barrier()
    pltpu.sync_copy(acc.at[pl.ds(s * ZCH, ZCH)], zbuf)
    pltpu.sync_copy(zbuf, out_hbm.at[c, pl.ds(s * ZCH, ZCH)])


@functools.partial(
    pl.kernel,
    mesh=_mesh,
    out_type=jax.ShapeDtypeStruct((NC, NR, H), jnp.float32),
    scratch_types=[
        pltpu.VMEM_SHARED((NR, H), jnp.float32),
        pltpu.VMEM((NCHH, KE), jnp.int32),
        pltpu.VMEM((NCHH, KE), jnp.int32),
        pltpu.VMEM((NCHH, KE), jnp.float32),
        pltpu.VMEM((2, KE, H), jnp.float32),
        pltpu.SemaphoreType.DMA,
        pltpu.SemaphoreType.DMA,
        pltpu.SemaphoreType.DMA,
        pltpu.SemaphoreType.DMA,
    ],
)
def _agg_call(s_hbm, row_hbm, col_hbm, ew_hbm, out_hbm, acc, rowv, colv, ewv,
              buf, gsem0, gsem1, ssem0, ssem1):
    c = lax.axis_index("c")
    s = lax.axis_index("s")
    wid = c * NS + s
    gsems = (gsem0, gsem1)
    ssems = (ssem0, ssem1)
    zero = jnp.zeros((16,), jnp.float32)

    def zr(r, _):
        for q in range(8):
            buf[0, r, pl.ds(q * 16, 16)] = zero
        return 0

    lax.fori_loop(0, KE, zr, 0)
    for (o, n) in _WB:
        pltpu.sync_copy(buf.at[0, pl.ds(0, n)],
                        acc.at[pl.ds(s * ZR + o, n)])
    plsc.subcore_barrier()

    for hp in range(2):
        pltpu.sync_copy(row_hbm.at[wid, pl.ds(hp * NCHH, NCHH)], rowv)
        pltpu.sync_copy(col_hbm.at[wid, pl.ds(hp * NCHH, NCHH)], colv)
        pltpu.sync_copy(ew_hbm.at[wid, pl.ds(hp * NCHH, NCHH)], ewv)

        def ch(j, _):
            pltpu.async_copy(s_hbm.at[rowv.at[j]], buf.at[0], gsems[0]).wait()

            def eb(q2, _):
                base = pl.multiple_of(q2 * 16, 16)
                wv = ewv[j, pl.ds(base, 16)]
                for l in range(16):
                    w = wv[l]
                    e = base + l
                    for q in range(8):
                        buf[0, e, pl.ds(q * 16, 16)] = (
                            buf[0, e, pl.ds(q * 16, 16)] * w)
                return 0

            lax.fori_loop(0, KE // 16, eb, 0)
            pltpu.sync_copy(buf.at[0], acc.at[colv.at[j]], add=True)
            return 0

        lax.fori_loop(0, NCHH, ch, 0)
    plsc.subcore_barrier()
    for (o, n) in _WB:
        pltpu.sync_copy(acc.at[pl.ds(s * ZR + o, n)], buf.at[0, pl.ds(0, n)])
        pltpu.sync_copy(buf.at[0, pl.ds(0, n)],
                        out_hbm.at[c, pl.ds(s * ZR + o, n)])


def _prep_body(deg_ref, x_ref, w_ref, s_ref, dinv_ref):
    deg = deg_ref[0, :, 0] + deg_ref[1, :, 0] + 1.0
    dinv = jnp.where(deg > 0, lax.rsqrt(deg), 0.0)
    dinv_ref[...] = dinv[:, None]
    s_ref[...] = jnp.dot(x_ref[...], w_ref[...],
                         preferred_element_type=jnp.float32) * dinv[:, None]


def _stats_body(a_ref, s_ref, dinv_ref, b_ref, t_ref, st_ref, *, relu_first):
    i = pl.program_id(0)
    o = (a_ref[0] + a_ref[1] + s_ref[...]) * dinv_ref[...] + b_ref[...]
    if relu_first:
        o = jnp.maximum(o, 0.0)
    t_ref[...] = o

    @pl.when(i == 0)
    def _():
        st_ref[...] = jnp.zeros_like(st_ref)

    st_ref[0:1] += jnp.sum(o, axis=0, keepdims=True)
    st_ref[1:2] += jnp.sum(o * o, axis=0, keepdims=True)


def _apply_body(t_ref, st_ref, g_ref, be_ref, w_ref, dinv_ref, o_ref, *,
                relu_after):
    mu = st_ref[0:1] * (1.0 / N)
    var = st_ref[1:2] * (1.0 / N) - mu * mu
    tn = g_ref[...] * (t_ref[...] - mu) / jnp.sqrt(var + EPS) + be_ref[...]
    if relu_after:
        tn = jnp.maximum(tn, 0.0)
    o_ref[...] = jnp.dot(tn, w_ref[...],
                         preferred_element_type=jnp.float32) * dinv_ref[...]


def _final_body(t_ref, st_ref, g_ref, be_ref, batch_ref, fw1_ref, fb1_ref,
                fw2_ref, fb2_ref, o_ref, acc_ref):
    i = pl.program_id(0)
    mu = st_ref[0:1] * (1.0 / N)
    var = st_ref[1:2] * (1.0 / N) - mu * mu
    hn = g_ref[...] * (t_ref[...] - mu) / jnp.sqrt(var + EPS) + be_ref[...]
    gid = lax.broadcasted_iota(jnp.int32, (G, RB), 0)
    onehot = (batch_ref[...][:, 0][None, :] == gid).astype(jnp.float32)

    @pl.when(i == 0)
    def _():
        acc_ref[...] = jnp.zeros_like(acc_ref)

    acc_ref[...] += jnp.dot(onehot, hn, preferred_element_type=jnp.float32)

    @pl.when(i == NBLK - 1)
    def _():
        pooled = jnp.maximum(acc_ref[...], 0.0)
        z = jnp.maximum(
            jnp.dot(pooled, fw1_ref[...], preferred_element_type=jnp.float32)
            + fb1_ref[...], 0.0)
        o_ref[...] = jnp.dot(z, fw2_ref[...],
                             preferred_element_type=jnp.float32) + fb2_ref[...]


_rowspec = pl.BlockSpec((RB, H), lambda i: (i, 0))
_vecspec = pl.BlockSpec((RB, 1), lambda i: (i, 0))
_wspec = pl.BlockSpec((H, H), lambda i: (0, 0))
_bspec = pl.BlockSpec((1, H), lambda i: (0, 0))
_stspec = pl.BlockSpec((2, H), lambda i: (0, 0))

_prep = pl.pallas_call(
    _prep_body,
    grid=(NBLK,),
    in_specs=[pl.BlockSpec((NC, RB, 1), lambda i: (0, i, 0)), _rowspec, _wspec],
    out_specs=[_rowspec, _vecspec],
    out_shape=[jax.ShapeDtypeStruct((N, H), jnp.float32),
               jax.ShapeDtypeStruct((N, 1), jnp.float32)],
)


def _stats(relu_first):
    return pl.pallas_call(
        functools.partial(_stats_body, relu_first=relu_first),
        grid=(NBLK,),
        in_specs=[pl.BlockSpec((NC, RB, H), lambda i: (0, i, 0)), _rowspec,
                  _vecspec, _bspec],
        out_specs=[_rowspec, _stspec],
        out_shape=[jax.ShapeDtypeStruct((N, H), jnp.float32),
                   jax.ShapeDtypeStruct((2, H), jnp.float32)],
    )


def _apply(relu_after):
    return pl.pallas_call(
        functools.partial(_apply_body, relu_after=relu_after),
        grid=(NBLK,),
        in_specs=[_rowspec, _stspec, _bspec, _bspec, _wspec, _vecspec],
        out_specs=_rowspec,
        out_shape=jax.ShapeDtypeStruct((N, H), jnp.float32),
    )


_final = pl.pallas_call(
    _final_body,
    grid=(NBLK,),
    in_specs=[_rowspec, _stspec, _bspec, _bspec,
              pl.BlockSpec((RB, 1), lambda i: (i, 0)), _wspec,
              _bspec, pl.BlockSpec((H, 20), lambda i: (0, 0)),
              pl.BlockSpec((1, 20), lambda i: (0, 0))],
    out_specs=pl.BlockSpec((G, 20), lambda i: (0, 0)),
    out_shape=jax.ShapeDtypeStruct((G, 20), jnp.float32),
    scratch_shapes=[pltpu.VMEM((G, H), jnp.float32)],
)

_ORDER = [(True, False), (True, False), (True, False), (False, True),
          (False, False)]


def kernel(x, edge_index, edge_weight, batch, params):
    p = params
    pad = EPAD - E
    row_p = jnp.concatenate(
        [edge_index[0], jnp.zeros((pad,), jnp.int32)]).reshape(NW, NCHE, KE)
    col_p = jnp.concatenate(
        [edge_index[1],
         jnp.full((pad,), NR - 1, jnp.int32)]).reshape(NW, NCHE, KE)
    ew_p = jnp.concatenate(
        [edge_weight, jnp.zeros((pad,), jnp.float32)]).reshape(NW, NCHE, KE)

    degp = _deg_call(col_p, ew_p)
    degp3 = degp.reshape(NC, NPAD, 1)
    s, dinv = _prep(degp3, x, p['W1'])

    out = None
    for li in range(1, 6):
        relu_first, relu_after = _ORDER[li - 1]
        aggp = _agg_call(s, row_p, col_p, ew_p)
        t, st = _stats(relu_first)(aggp, s, dinv, p['b%d' % li][None, :])
        if li < 5:
            s = _apply(relu_after)(t, st, p['g%d' % li][None, :],
                                   p['be%d' % li][None, :], p['W%d' % (li + 1)],
                                   dinv)
        else:
            out = _final(t, st, p['g%d' % li][None, :], p['be%d' % li][None, :],
                         batch.reshape(N, 1), p['fW1'], p['fb1'][None, :],
                         p['fW2'], p['fb2'][None, :])
    return out


# speedup vs baseline: 1.0002x; 1.0002x over previous
import functools

import jax
import jax.numpy as jnp
from jax import lax
from jax.experimental import pallas as pl
from jax.experimental.pallas import tpu as pltpu
from jax.experimental.pallas import tpu_sc as plsc

N = 10000
E = 320000
H = 128
G = 64
EPS = 1e-5

NC = 2
NS = 16
NW = NC * NS
KE = 128
NCHE = 80
EPT = NCHE * KE
EPAD = NW * EPT
NPAD = 10240
ZCH = NPAD // NS
NR = 10112
ZR = NR // NS
NCHH = NCHE // 2
_WB = [(0, 128), (128, 128), (256, 128), (384, 128), (512, 120)]

RB = 1000
NBLK = N // RB

_mesh = plsc.VectorSubcoreMesh(core_axis_name="c", subcore_axis_name="s")


@functools.partial(
    pl.kernel,
    mesh=_mesh,
    out_type=jax.ShapeDtypeStruct((NC, NPAD), jnp.float32),
    scratch_types=[
        pltpu.VMEM_SHARED((NPAD,), jnp.float32),
        pltpu.VMEM((NCHE, KE), jnp.int32),
        pltpu.VMEM((NCHE, KE), jnp.float32),
        pltpu.VMEM((ZCH,), jnp.float32),
    ],
)
def _deg_call(col_hbm, ew_hbm, out_hbm, acc, colv, ewv, zbuf):
    c = lax.axis_index("c")
    s = lax.axis_index("s")
    wid = c * NS + s
    zero = jnp.zeros((16,), jnp.float32)

    def zb(i, _):
        zbuf[pl.ds(pl.multiple_of(i * 16, 16), 16)] = zero
        return 0

    lax.fori_loop(0, ZCH // 16, zb, 0)
    pltpu.sync_copy(zbuf, acc.at[pl.ds(s * ZCH, ZCH)])
    pltpu.sync_copy(col_hbm.at[wid], colv)
    pltpu.sync_copy(ew_hbm.at[wid], ewv)
    plsc.subcore_barrier()

    def ch(j, _):
        pltpu.sync_copy(ewv.at[j], acc.at[colv.at[j]], add=True)
        return 0

    lax.fori_loop(0, NCHE, ch, 0)
    plsc.subcore_barrier()
    pltpu.sync_copy(acc.at[pl.ds(s * ZCH, ZCH)], zbuf)
    pltpu.sync_copy(zbuf, out_hbm.at[c, pl.ds(s * ZCH, ZCH)])


@functools.partial(
    pl.kernel,
    mesh=_mesh,
    out_type=jax.ShapeDtypeStruct((NC, NR, H), jnp.float32),
    scratch_types=[
        pltpu.VMEM_SHARED((NR, H), jnp.float32),
        pltpu.VMEM((NCHH, KE), jnp.int32),
        pltpu.VMEM((NCHH, KE), jnp.int32),
        pltpu.VMEM((NCHH, KE), jnp.float32),
        pltpu.VMEM((KE, H), jnp.float32),
        pltpu.VMEM((KE, H), jnp.float32),
        pltpu.SemaphoreType.DMA,
        pltpu.SemaphoreType.DMA,
    ],
)
def _agg_call(s_hbm, row_hbm, col_hbm, ew_hbm, out_hbm, acc, rowv, colv, ewv,
              buf0, buf1, gsem0, gsem1):
    c = lax.axis_index("c")
    s = lax.axis_index("s")
    wid = c * NS + s
    zero = jnp.zeros((16,), jnp.float32)

    def zr(r, _):
        for q in range(8):
            buf0[r, pl.ds(q * 16, 16)] = zero
        return 0

    lax.fori_loop(0, KE, zr, 0)
    for (o, n) in _WB:
        pltpu.sync_copy(buf0.at[pl.ds(0, n)], acc.at[pl.ds(s * ZR + o, n)])
    plsc.subcore_barrier()

    for hp in range(2):
        pltpu.sync_copy(row_hbm.at[wid, pl.ds(hp * NCHH, NCHH)], rowv)
        pltpu.sync_copy(col_hbm.at[wid, pl.ds(hp * NCHH, NCHH)], colv)
        pltpu.sync_copy(ew_hbm.at[wid, pl.ds(hp * NCHH, NCHH)], ewv)

        def ch(j, _):
            pltpu.async_copy(s_hbm.at[rowv.at[j]], buf0, gsem0).wait()

            def eb(q2, _):
                base = pl.multiple_of(q2 * 16, 16)
                wv = ewv[j, pl.ds(base, 16)]
                for l in range(16):
                    w = wv[l]
                    e = base + l
                    for q in range(8):
                        buf0[e, pl.ds(q * 16, 16)] = (
                            buf0[e, pl.ds(q * 16, 16)] * w)
                return 0

            lax.fori_loop(0, KE // 16, eb, 0)
            pltpu.sync_copy(buf0, acc.at[colv.at[j]], add=True)
            return 0

        lax.fori_loop(0, NCHH, ch, 0)
    plsc.subcore_barrier()
    for (o, n) in _WB:
        pltpu.sync_copy(acc.at[pl.ds(s * ZR + o, n)], buf0.at[pl.ds(0, n)])
        pltpu.sync_copy(buf0.at[pl.ds(0, n)],
                        out_hbm.at[c, pl.ds(s * ZR + o, n)])


def _prep_body(deg_ref, x_ref, w_ref, s_ref, dinv_ref):
    deg = deg_ref[0, :, 0] + deg_ref[1, :, 0] + 1.0
    dinv = jnp.where(deg > 0, lax.rsqrt(deg), 0.0)
    dinv_ref[...] = dinv[:, None]
    s_ref[...] = jnp.dot(x_ref[...], w_ref[...],
                         preferred_element_type=jnp.float32) * dinv[:, None]


def _stats_body(a_ref, s_ref, dinv_ref, b_ref, t_ref, st_ref, *, relu_first):
    i = pl.program_id(0)
    o = (a_ref[0] + a_ref[1] + s_ref[...]) * dinv_ref[...] + b_ref[...]
    if relu_first:
        o = jnp.maximum(o, 0.0)
    t_ref[...] = o

    @pl.when(i == 0)
    def _():
        st_ref[...] = jnp.zeros_like(st_ref)

    st_ref[0:1] += jnp.sum(o, axis=0, keepdims=True)
    st_ref[1:2] += jnp.sum(o * o, axis=0, keepdims=True)


def _apply_body(t_ref, st_ref, g_ref, be_ref, w_ref, dinv_ref, o_ref, *,
                relu_after):
    mu = st_ref[0:1] * (1.0 / N)
    var = st_ref[1:2] * (1.0 / N) - mu * mu
    tn = g_ref[...] * (t_ref[...] - mu) / jnp.sqrt(var + EPS) + be_ref[...]
    if relu_after:
        tn = jnp.maximum(tn, 0.0)
    o_ref[...] = jnp.dot(tn, w_ref[...],
                         preferred_element_type=jnp.float32) * dinv_ref[...]


def _final_body(t_ref, st_ref, g_ref, be_ref, batch_ref, fw1_ref, fb1_ref,
                fw2_ref, fb2_ref, o_ref, acc_ref):
    i = pl.program_id(0)
    mu = st_ref[0:1] * (1.0 / N)
    var = st_ref[1:2] * (1.0 / N) - mu * mu
    hn = g_ref[...] * (t_ref[...] - mu) / jnp.sqrt(var + EPS) + be_ref[...]
    gid = lax.broadcasted_iota(jnp.int32, (G, RB), 0)
    onehot = (batch_ref[...][:, 0][None, :] == gid).astype(jnp.float32)

    @pl.when(i == 0)
    def _():
        acc_ref[...] = jnp.zeros_like(acc_ref)

    acc_ref[...] += jnp.dot(onehot, hn, preferred_element_type=jnp.float32)

    @pl.when(i == NBLK - 1)
    def _():
        pooled = jnp.maximum(acc_ref[...], 0.0)
        z = jnp.maximum(
            jnp.dot(pooled, fw1_ref[...], preferred_element_type=jnp.float32)
            + fb1_ref[...], 0.0)
        o_ref[...] = jnp.dot(z, fw2_ref[...],
                             preferred_element_type=jnp.float32) + fb2_ref[...]


_rowspec = pl.BlockSpec((RB, H), lambda i: (i, 0))
_vecspec = pl.BlockSpec((RB, 1), lambda i: (i, 0))
_wspec = pl.BlockSpec((H, H), lambda i: (0, 0))
_bspec = pl.BlockSpec((1, H), lambda i: (0, 0))
_stspec = pl.BlockSpec((2, H), lambda i: (0, 0))

_prep = pl.pallas_call(
    _prep_body,
    grid=(NBLK,),
    in_specs=[pl.BlockSpec((NC, RB, 1), lambda i: (0, i, 0)), _rowspec, _wspec],
    out_specs=[_rowspec, _vecspec],
    out_shape=[jax.ShapeDtypeStruct((N, H), jnp.float32),
               jax.ShapeDtypeStruct((N, 1), jnp.float32)],
)


def _stats(relu_first):
    return pl.pallas_call(
        functools.partial(_stats_body, relu_first=relu_first),
        grid=(NBLK,),
        in_specs=[pl.BlockSpec((NC, RB, H), lambda i: (0, i, 0)), _rowspec,
                  _vecspec, _bspec],
        out_specs=[_rowspec, _stspec],
        out_shape=[jax.ShapeDtypeStruct((N, H), jnp.float32),
                   jax.ShapeDtypeStruct((2, H), jnp.float32)],
    )


def _apply(relu_after):
    return pl.pallas_call(
        functools.partial(_apply_body, relu_after=relu_after),
        grid=(NBLK,),
        in_specs=[_rowspec, _stspec, _bspec, _bspec, _wspec, _vecspec],
        out_specs=_rowspec,
        out_shape=jax.ShapeDtypeStruct((N, H), jnp.float32),
    )


_final = pl.pallas_call(
    _final_body,
    grid=(NBLK,),
    in_specs=[_rowspec, _stspec, _bspec, _bspec,
              pl.BlockSpec((RB, 1), lambda i: (i, 0)), _wspec,
              _bspec, pl.BlockSpec((H, 20), lambda i: (0, 0)),
              pl.BlockSpec((1, 20), lambda i: (0, 0))],
    out_specs=pl.BlockSpec((G, 20), lambda i: (0, 0)),
    out_shape=jax.ShapeDtypeStruct((G, 20), jnp.float32),
    scratch_shapes=[pltpu.VMEM((G, H), jnp.float32)],
)

_ORDER = [(True, False), (True, False), (True, False), (False, True),
          (False, False)]


def kernel(x, edge_index, edge_weight, batch, params):
    p = params
    pad = EPAD - E
    row_p = jnp.concatenate(
        [edge_index[0], jnp.zeros((pad,), jnp.int32)]).reshape(NW, NCHE, KE)
    col_p = jnp.concatenate(
        [edge_index[1],
         jnp.full((pad,), NR - 1, jnp.int32)]).reshape(NW, NCHE, KE)
    ew_p = jnp.concatenate(
        [edge_weight, jnp.zeros((pad,), jnp.float32)]).reshape(NW, NCHE, KE)

    degp = _deg_call(col_p, ew_p)
    degp3 = degp.reshape(NC, NPAD, 1)
    s, dinv = _prep(degp3, x, p['W1'])

    out = None
    for li in range(1, 6):
        relu_first, relu_after = _ORDER[li - 1]
        aggp = _agg_call(s, row_p, col_p, ew_p)
        t, st = _stats(relu_first)(aggp, s, dinv, p['b%d' % li][None, :])
        if li < 5:
            s = _apply(relu_after)(t, st, p['g%d' % li][None, :],
                                   p['be%d' % li][None, :], p['W%d' % (li + 1)],
                                   dinv)
        else:
            out = _final(t, st, p['g%d' % li][None, :], p['be%d' % li][None, :],
                         batch.reshape(N, 1), p['fW1'], p['fb1'][None, :],
                         p['fW2'], p['fb2'][None, :])
    return out


# speedup vs baseline: 1.4290x; 1.4287x over previous
import functools

import jax
import jax.numpy as jnp
from jax import lax
from jax.experimental import pallas as pl
from jax.experimental.pallas import tpu as pltpu
from jax.experimental.pallas import tpu_sc as plsc

N = 10000
E = 320000
H = 128
HH = H // 2
G = 64
EPS = 1e-5

NC = 2
NS = 16
NW = NC * NS
KE = 64
NCH = 160
EPT = NCH * KE
NCHH = NCH // 4
NPAD = 10240
ZCH = NPAD // NS
NR = 10112
ZR = NR // NS
_WB = [(i * 64, 64) for i in range(9)] + [(576, 56)]
NBUF = 4

RB = 1000
NBLK = N // RB

_mesh = plsc.VectorSubcoreMesh(core_axis_name="c", subcore_axis_name="s")


@functools.partial(
    pl.kernel,
    mesh=_mesh,
    out_type=jax.ShapeDtypeStruct((NC, NPAD), jnp.float32),
    scratch_types=[
        pltpu.VMEM_SHARED((NPAD,), jnp.float32),
        pltpu.VMEM((NCH, KE), jnp.int32),
        pltpu.VMEM((NCH, KE), jnp.float32),
        pltpu.VMEM((ZCH,), jnp.float32),
    ],
)
def _deg_call(col_hbm, ew_hbm, out_hbm, acc, colv, ewv, zbuf):
    c = lax.axis_index("c")
    s = lax.axis_index("s")
    wid = c * NS + s
    zero = jnp.zeros((16,), jnp.float32)

    def zb(i, _):
        zbuf[pl.ds(pl.multiple_of(i * 16, 16), 16)] = zero
        return 0

    lax.fori_loop(0, ZCH // 16, zb, 0)
    pltpu.sync_copy(zbuf, acc.at[pl.ds(s * ZCH, ZCH)])
    pltpu.sync_copy(col_hbm.at[wid], colv)
    pltpu.sync_copy(ew_hbm.at[wid], ewv)
    plsc.subcore_barrier()

    def ch(j, _):
        pltpu.sync_copy(ewv.at[j], acc.at[colv.at[j]], add=True)
        return 0

    lax.fori_loop(0, NCH, ch, 0)
    plsc.subcore_barrier()
    pltpu.sync_copy(acc.at[pl.ds(s * ZCH, ZCH)], zbuf)
    pltpu.sync_copy(zbuf, out_hbm.at[c, pl.ds(s * ZCH, ZCH)])


@functools.partial(
    pl.kernel,
    mesh=_mesh,
    out_type=jax.ShapeDtypeStruct((NC, NR, H), jnp.float32),
    scratch_types=[
        pltpu.VMEM_SHARED((NR, H), jnp.float32),
        pltpu.VMEM((NCHH, KE), jnp.int32),
        pltpu.VMEM((NCHH, KE), jnp.int32),
        pltpu.VMEM((NCHH, KE), jnp.float32),
        pltpu.VMEM((KE, H), jnp.float32),
        pltpu.VMEM((KE, H), jnp.float32),
        pltpu.VMEM((KE, H), jnp.float32),
        pltpu.VMEM((KE, H), jnp.float32),
        pltpu.SemaphoreType.DMA,
        pltpu.SemaphoreType.DMA,
        pltpu.SemaphoreType.DMA,
        pltpu.SemaphoreType.DMA,
    ],
)
def _agg_call(s_hbm, row_hbm, col_hbm, ew_hbm, out_hbm, acc, rowv, colv, ewv,
              buf0, buf1, buf2, buf3, gsem0, gsem1, gsem2, gsem3):
    c = lax.axis_index("c")
    s = lax.axis_index("s")
    wid = c * NS + s
    bufs = (buf0, buf1, buf2, buf3)
    gsems = (gsem0, gsem1, gsem2, gsem3)
    zero = jnp.zeros((16,), jnp.float32)

    def zr(r, _):
        for q in range(H // 16):
            buf0[r, pl.ds(q * 16, 16)] = zero
        return 0

    lax.fori_loop(0, KE, zr, 0)
    for (o, n) in _WB:
        pltpu.sync_copy(buf0.at[pl.ds(0, n)], acc.at[pl.ds(s * ZR + o, n)])
    plsc.subcore_barrier()

    for hp in range(4):
        pltpu.sync_copy(row_hbm.at[wid, pl.ds(hp * NCHH, NCHH)], rowv)
        pltpu.sync_copy(col_hbm.at[wid, pl.ds(hp * NCHH, NCHH)], colv)
        pltpu.sync_copy(ew_hbm.at[wid, pl.ds(hp * NCHH, NCHH)], ewv)

        for b in range(NBUF):
            pltpu.async_copy(s_hbm.at[rowv.at[b]], bufs[b], gsems[b])

        def ch4(jp, _):
            for b in range(NBUF):
                j = jp * NBUF + b
                pltpu.make_async_copy(s_hbm.at[rowv.at[j]], bufs[b],
                                      gsems[b]).wait()

                def eb(q2, _):
                    base = pl.multiple_of(q2 * 16, 16)
                    wv = ewv[j, pl.ds(base, 16)]
                    for l in range(16):
                        w = wv[l]
                        e = base + l
                        for q in range(H // 16):
                            bufs[b][e, pl.ds(q * 16, 16)] = (
                                bufs[b][e, pl.ds(q * 16, 16)] * w)
                    return 0

                lax.fori_loop(0, KE // 16, eb, 0)
                pltpu.sync_copy(bufs[b], acc.at[colv.at[j]], add=True)

                @pl.when(jp < NCHH // NBUF - 1)
                def _():
                    pltpu.async_copy(s_hbm.at[rowv.at[j + NBUF]],
                                     bufs[b], gsems[b])
            return 0

        lax.fori_loop(0, NCHH // NBUF, ch4, 0)
    plsc.subcore_barrier()
    for (o, n) in _WB:
        pltpu.sync_copy(acc.at[pl.ds(s * ZR + o, n)], buf0.at[pl.ds(0, n)])
        pltpu.sync_copy(buf0.at[pl.ds(0, n)],
                        out_hbm.at[c, pl.ds(s * ZR + o, n)])


def _prep_body(deg_ref, x_ref, w_ref, s_ref, dinv_ref):
    deg = deg_ref[0, :, 0] + deg_ref[1, :, 0] + 1.0
    dinv = jnp.where(deg > 0, lax.rsqrt(deg), 0.0)
    dinv_ref[...] = dinv[:, None]
    s_ref[...] = jnp.dot(x_ref[...], w_ref[...],
                         preferred_element_type=jnp.float32) * dinv[:, None]


def _stats_body(a_ref, s_ref, dinv_ref, b_ref, t_ref, st_ref, *, relu_first):
    i = pl.program_id(0)
    o = (a_ref[0] + a_ref[1] + s_ref[...]) * dinv_ref[...] + b_ref[...]
    if relu_first:
        o = jnp.maximum(o, 0.0)
    t_ref[...] = o

    @pl.when(i == 0)
    def _():
        st_ref[...] = jnp.zeros_like(st_ref)

    st_ref[0:1] += jnp.sum(o, axis=0, keepdims=True)
    st_ref[1:2] += jnp.sum(o * o, axis=0, keepdims=True)


def _apply_body(t_ref, st_ref, g_ref, be_ref, w_ref, dinv_ref, o_ref, *,
                relu_after):
    mu = st_ref[0:1] * (1.0 / N)
    var = st_ref[1:2] * (1.0 / N) - mu * mu
    tn = g_ref[...] * (t_ref[...] - mu) / jnp.sqrt(var + EPS) + be_ref[...]
    if relu_after:
        tn = jnp.maximum(tn, 0.0)
    o_ref[...] = jnp.dot(tn, w_ref[...],
                         preferred_element_type=jnp.float32) * dinv_ref[...]


def _final_body(t_ref, st_ref, g_ref, be_ref, batch_ref, fw1_ref, fb1_ref,
                fw2_ref, fb2_ref, o_ref, acc_ref):
    i = pl.program_id(0)
    mu = st_ref[0:1] * (1.0 / N)
    var = st_ref[1:2] * (1.0 / N) - mu * mu
    hn = g_ref[...] * (t_ref[...] - mu) / jnp.sqrt(var + EPS) + be_ref[...]
    gid = lax.broadcasted_iota(jnp.int32, (G, RB), 0)
    onehot = (batch_ref[...][:, 0][None, :] == gid).astype(jnp.float32)

    @pl.when(i == 0)
    def _():
        acc_ref[...] = jnp.zeros_like(acc_ref)

    acc_ref[...] += jnp.dot(onehot, hn, preferred_element_type=jnp.float32)

    @pl.when(i == NBLK - 1)
    def _():
        pooled = jnp.maximum(acc_ref[...], 0.0)
        z = jnp.maximum(
            jnp.dot(pooled, fw1_ref[...], preferred_element_type=jnp.float32)
            + fb1_ref[...], 0.0)
        o_ref[...] = jnp.dot(z, fw2_ref[...],
                             preferred_element_type=jnp.float32) + fb2_ref[...]


_rowspec = pl.BlockSpec((RB, H), lambda i: (i, 0))
_aggspec = pl.BlockSpec((NC, RB, H), lambda i: (0, i, 0))
_vecspec = pl.BlockSpec((RB, 1), lambda i: (i, 0))
_wspec = pl.BlockSpec((H, H), lambda i: (0, 0))
_bspec = pl.BlockSpec((1, H), lambda i: (0, 0))
_stspec = pl.BlockSpec((2, H), lambda i: (0, 0))

_prep = pl.pallas_call(
    _prep_body,
    grid=(NBLK,),
    in_specs=[pl.BlockSpec((NC, RB, 1), lambda i: (0, i, 0)), _rowspec, _wspec],
    out_specs=[_rowspec, _vecspec],
    out_shape=[jax.ShapeDtypeStruct((N, H), jnp.float32),
               jax.ShapeDtypeStruct((N, 1), jnp.float32)],
)


def _stats(relu_first):
    return pl.pallas_call(
        functools.partial(_stats_body, relu_first=relu_first),
        grid=(NBLK,),
        in_specs=[_aggspec, _rowspec, _vecspec, _bspec],
        out_specs=[_rowspec, _stspec],
        out_shape=[jax.ShapeDtypeStruct((N, H), jnp.float32),
                   jax.ShapeDtypeStruct((2, H), jnp.float32)],
    )


def _apply(relu_after):
    return pl.pallas_call(
        functools.partial(_apply_body, relu_after=relu_after),
        grid=(NBLK,),
        in_specs=[_rowspec, _stspec, _bspec, _bspec, _wspec, _vecspec],
        out_specs=_rowspec,
        out_shape=jax.ShapeDtypeStruct((N, H), jnp.float32),
    )


_final = pl.pallas_call(
    _final_body,
    grid=(NBLK,),
    in_specs=[_rowspec, _stspec, _bspec, _bspec,
              pl.BlockSpec((RB, 1), lambda i: (i, 0)), _wspec,
              _bspec, pl.BlockSpec((H, 20), lambda i: (0, 0)),
              pl.BlockSpec((1, 20), lambda i: (0, 0))],
    out_specs=pl.BlockSpec((G, 20), lambda i: (0, 0)),
    out_shape=jax.ShapeDtypeStruct((G, 20), jnp.float32),
    scratch_shapes=[pltpu.VMEM((G, H), jnp.float32)],
)

_ORDER = [(True, False), (True, False), (True, False), (False, True),
          (False, False)]


def kernel(x, edge_index, edge_weight, batch, params):
    p = params
    ept = E // NW
    padw = ((0, 0), (0, EPT - ept))
    row_p = jnp.pad(edge_index[0].reshape(NW, ept), padw).reshape(NW, NCH, KE)
    col_p = jnp.pad(edge_index[1].reshape(NW, ept), padw,
                    constant_values=NR - 1).reshape(NW, NCH, KE)
    ew_p = jnp.pad(edge_weight.reshape(NW, ept), padw).reshape(NW, NCH, KE)

    degp = _deg_call(col_p, ew_p)
    degp3 = degp.reshape(NC, NPAD, 1)
    s, dinv = _prep(degp3, x, p['W1'])

    out = None
    for li in range(1, 6):
        relu_first, relu_after = _ORDER[li - 1]
        aggp = _agg_call(s, row_p, col_p, ew_p)
        t, st = _stats(relu_first)(aggp, s, dinv, p['b%d' % li][None, :])
        if li < 5:
            s = _apply(relu_after)(t, st, p['g%d' % li][None, :],
                                   p['be%d' % li][None, :], p['W%d' % (li + 1)],
                                   dinv)
        else:
            out = _final(t, st, p['g%d' % li][None, :], p['be%d' % li][None, :],
                         batch.reshape(N, 1), p['fW1'], p['fb1'][None, :],
                         p['fW2'], p['fb2'][None, :])
    return out


# speedup vs baseline: 1.4661x; 1.0259x over previous
import functools

import jax
import jax.numpy as jnp
from jax import lax
from jax.experimental import pallas as pl
from jax.experimental.pallas import tpu as pltpu
from jax.experimental.pallas import tpu_sc as plsc

N = 10000
E = 320000
H = 128
HH = H // 2
G = 64
EPS = 1e-5

NC = 2
NS = 16
NW = NC * NS
KE = 128
NCH = 80
EPT = NCH * KE
NCHH = NCH // 2
NPAD = 10240
ZCH = NPAD // NS
NR = 10112
ZR = NR // NS
_WB = [(i * 128, 128) for i in range(4)] + [(512, 120)]
NBUF = 2

RB = 1000
NBLK = N // RB

_mesh = plsc.VectorSubcoreMesh(core_axis_name="c", subcore_axis_name="s")


@functools.partial(
    pl.kernel,
    mesh=_mesh,
    out_type=jax.ShapeDtypeStruct((NC, NPAD), jnp.float32),
    scratch_types=[
        pltpu.VMEM_SHARED((NPAD,), jnp.float32),
        pltpu.VMEM((NCH, KE), jnp.int32),
        pltpu.VMEM((NCH, KE), jnp.float32),
        pltpu.VMEM((ZCH,), jnp.float32),
    ],
)
def _deg_call(col_hbm, ew_hbm, out_hbm, acc, colv, ewv, zbuf):
    c = lax.axis_index("c")
    s = lax.axis_index("s")
    wid = c * NS + s
    zero = jnp.zeros((16,), jnp.float32)

    def zb(i, _):
        zbuf[pl.ds(pl.multiple_of(i * 16, 16), 16)] = zero
        return 0

    lax.fori_loop(0, ZCH // 16, zb, 0)
    pltpu.sync_copy(zbuf, acc.at[pl.ds(s * ZCH, ZCH)])
    pltpu.sync_copy(col_hbm.at[wid], colv)
    pltpu.sync_copy(ew_hbm.at[wid], ewv)
    plsc.subcore_barrier()

    def ch(j, _):
        pltpu.sync_copy(ewv.at[j], acc.at[colv.at[j]], add=True)
        return 0

    lax.fori_loop(0, NCH, ch, 0)
    plsc.subcore_barrier()
    pltpu.sync_copy(acc.at[pl.ds(s * ZCH, ZCH)], zbuf)
    pltpu.sync_copy(zbuf, out_hbm.at[c, pl.ds(s * ZCH, ZCH)])


@functools.partial(
    pl.kernel,
    mesh=_mesh,
    out_type=jax.ShapeDtypeStruct((NC, NR, H), jnp.float32),
    scratch_types=[
        pltpu.VMEM_SHARED((NR, H), jnp.float32),
        pltpu.VMEM((NCHH, KE), jnp.int32),
        pltpu.VMEM((NCHH, KE), jnp.int32),
        pltpu.VMEM((NCHH, KE), jnp.float32),
        pltpu.VMEM((KE, H), jnp.float32),
        pltpu.VMEM((KE, H), jnp.float32),
        pltpu.SemaphoreType.DMA,
        pltpu.SemaphoreType.DMA,
    ],
)
def _agg_call(s_hbm, row_hbm, col_hbm, ew_hbm, out_hbm, acc, rowv, colv, ewv,
              buf0, buf1, gsem0, gsem1):
    c = lax.axis_index("c")
    s = lax.axis_index("s")
    wid = c * NS + s
    bufs = (buf0, buf1)
    gsems = (gsem0, gsem1)
    zero = jnp.zeros((16,), jnp.float32)

    def zr(r, _):
        for q in range(H // 16):
            buf0[r, pl.ds(q * 16, 16)] = zero
        return 0

    lax.fori_loop(0, KE, zr, 0)
    for (o, n) in _WB:
        pltpu.sync_copy(buf0.at[pl.ds(0, n)], acc.at[pl.ds(s * ZR + o, n)])
    plsc.subcore_barrier()

    for hp in range(2):
        pltpu.sync_copy(row_hbm.at[wid, pl.ds(hp * NCHH, NCHH)], rowv)
        pltpu.sync_copy(col_hbm.at[wid, pl.ds(hp * NCHH, NCHH)], colv)
        pltpu.sync_copy(ew_hbm.at[wid, pl.ds(hp * NCHH, NCHH)], ewv)

        for b in range(NBUF):
            pltpu.async_copy(s_hbm.at[rowv.at[b]], bufs[b], gsems[b])

        def ch4(jp, _):
            for b in range(NBUF):
                j = jp * NBUF + b
                pltpu.make_async_copy(s_hbm.at[rowv.at[j]], bufs[b],
                                      gsems[b]).wait()

                def eb(q2, _):
                    base = pl.multiple_of(q2 * 16, 16)
                    wv = ewv[j, pl.ds(base, 16)]
                    for l in range(16):
                        w = wv[l]
                        e = base + l
                        for q in range(H // 16):
                            bufs[b][e, pl.ds(q * 16, 16)] = (
                                bufs[b][e, pl.ds(q * 16, 16)] * w)
                    return 0

                lax.fori_loop(0, KE // 16, eb, 0)
                pltpu.sync_copy(bufs[b], acc.at[colv.at[j]], add=True)

                @pl.when(jp < NCHH // NBUF - 1)
                def _():
                    pltpu.async_copy(s_hbm.at[rowv.at[j + NBUF]],
                                     bufs[b], gsems[b])
            return 0

        lax.fori_loop(0, NCHH // NBUF, ch4, 0)
    plsc.subcore_barrier()
    for (o, n) in _WB:
        pltpu.sync_copy(acc.at[pl.ds(s * ZR + o, n)], buf0.at[pl.ds(0, n)])
        pltpu.sync_copy(buf0.at[pl.ds(0, n)],
                        out_hbm.at[c, pl.ds(s * ZR + o, n)])


def _prep_body(deg_ref, x_ref, w_ref, s_ref, dinv_ref):
    deg = deg_ref[0, :, 0] + deg_ref[1, :, 0] + 1.0
    dinv = jnp.where(deg > 0, lax.rsqrt(deg), 0.0)
    dinv_ref[...] = dinv[:, None]
    s_ref[...] = jnp.dot(x_ref[...], w_ref[...],
                         preferred_element_type=jnp.float32) * dinv[:, None]


def _stats_body(a_ref, s_ref, dinv_ref, b_ref, t_ref, st_ref, *, relu_first):
    i = pl.program_id(0)
    o = (a_ref[0] + a_ref[1] + s_ref[...]) * dinv_ref[...] + b_ref[...]
    if relu_first:
        o = jnp.maximum(o, 0.0)
    t_ref[...] = o

    @pl.when(i == 0)
    def _():
        st_ref[...] = jnp.zeros_like(st_ref)

    st_ref[0:1] += jnp.sum(o, axis=0, keepdims=True)
    st_ref[1:2] += jnp.sum(o * o, axis=0, keepdims=True)


def _apply_body(t_ref, st_ref, g_ref, be_ref, w_ref, dinv_ref, o_ref, *,
                relu_after):
    mu = st_ref[0:1] * (1.0 / N)
    var = st_ref[1:2] * (1.0 / N) - mu * mu
    tn = g_ref[...] * (t_ref[...] - mu) / jnp.sqrt(var + EPS) + be_ref[...]
    if relu_after:
        tn = jnp.maximum(tn, 0.0)
    o_ref[...] = jnp.dot(tn, w_ref[...],
                         preferred_element_type=jnp.float32) * dinv_ref[...]


def _final_body(t_ref, st_ref, g_ref, be_ref, batch_ref, fw1_ref, fb1_ref,
                fw2_ref, fb2_ref, o_ref, acc_ref):
    i = pl.program_id(0)
    mu = st_ref[0:1] * (1.0 / N)
    var = st_ref[1:2] * (1.0 / N) - mu * mu
    hn = g_ref[...] * (t_ref[...] - mu) / jnp.sqrt(var + EPS) + be_ref[...]
    gid = lax.broadcasted_iota(jnp.int32, (G, RB), 0)
    onehot = (batch_ref[...][:, 0][None, :] == gid).astype(jnp.float32)

    @pl.when(i == 0)
    def _():
        acc_ref[...] = jnp.zeros_like(acc_ref)

    acc_ref[...] += jnp.dot(onehot, hn, preferred_element_type=jnp.float32)

    @pl.when(i == NBLK - 1)
    def _():
        pooled = jnp.maximum(acc_ref[...], 0.0)
        z = jnp.maximum(
            jnp.dot(pooled, fw1_ref[...], preferred_element_type=jnp.float32)
            + fb1_ref[...], 0.0)
        o_ref[...] = jnp.dot(z, fw2_ref[...],
                             preferred_element_type=jnp.float32) + fb2_ref[...]


_rowspec = pl.BlockSpec((RB, H), lambda i: (i, 0))
_aggspec = pl.BlockSpec((NC, RB, H), lambda i: (0, i, 0))
_vecspec = pl.BlockSpec((RB, 1), lambda i: (i, 0))
_wspec = pl.BlockSpec((H, H), lambda i: (0, 0))
_bspec = pl.BlockSpec((1, H), lambda i: (0, 0))
_stspec = pl.BlockSpec((2, H), lambda i: (0, 0))

_prep = pl.pallas_call(
    _prep_body,
    grid=(NBLK,),
    in_specs=[pl.BlockSpec((NC, RB, 1), lambda i: (0, i, 0)), _rowspec, _wspec],
    out_specs=[_rowspec, _vecspec],
    out_shape=[jax.ShapeDtypeStruct((N, H), jnp.float32),
               jax.ShapeDtypeStruct((N, 1), jnp.float32)],
)


def _stats(relu_first):
    return pl.pallas_call(
        functools.partial(_stats_body, relu_first=relu_first),
        grid=(NBLK,),
        in_specs=[_aggspec, _rowspec, _vecspec, _bspec],
        out_specs=[_rowspec, _stspec],
        out_shape=[jax.ShapeDtypeStruct((N, H), jnp.float32),
                   jax.ShapeDtypeStruct((2, H), jnp.float32)],
    )


def _apply(relu_after):
    return pl.pallas_call(
        functools.partial(_apply_body, relu_after=relu_after),
        grid=(NBLK,),
        in_specs=[_rowspec, _stspec, _bspec, _bspec, _wspec, _vecspec],
        out_specs=_rowspec,
        out_shape=jax.ShapeDtypeStruct((N, H), jnp.float32),
    )


_final = pl.pallas_call(
    _final_body,
    grid=(NBLK,),
    in_specs=[_rowspec, _stspec, _bspec, _bspec,
              pl.BlockSpec((RB, 1), lambda i: (i, 0)), _wspec,
              _bspec, pl.BlockSpec((H, 20), lambda i: (0, 0)),
              pl.BlockSpec((1, 20), lambda i: (0, 0))],
    out_specs=pl.BlockSpec((G, 20), lambda i: (0, 0)),
    out_shape=jax.ShapeDtypeStruct((G, 20), jnp.float32),
    scratch_shapes=[pltpu.VMEM((G, H), jnp.float32)],
)

_ORDER = [(True, False), (True, False), (True, False), (False, True),
          (False, False)]


def kernel(x, edge_index, edge_weight, batch, params):
    p = params
    ept = E // NW
    padw = ((0, 0), (0, EPT - ept))
    row_p = jnp.pad(edge_index[0].reshape(NW, ept), padw).reshape(NW, NCH, KE)
    col_p = jnp.pad(edge_index[1].reshape(NW, ept), padw,
                    constant_values=NR - 1).reshape(NW, NCH, KE)
    ew_p = jnp.pad(edge_weight.reshape(NW, ept), padw).reshape(NW, NCH, KE)

    degp = _deg_call(col_p, ew_p)
    degp3 = degp.reshape(NC, NPAD, 1)
    s, dinv = _prep(degp3, x, p['W1'])

    out = None
    for li in range(1, 6):
        relu_first, relu_after = _ORDER[li - 1]
        aggp = _agg_call(s, row_p, col_p, ew_p)
        t, st = _stats(relu_first)(aggp, s, dinv, p['b%d' % li][None, :])
        if li < 5:
            s = _apply(relu_after)(t, st, p['g%d' % li][None, :],
                                   p['be%d' % li][None, :], p['W%d' % (li + 1)],
                                   dinv)
        else:
            out = _final(t, st, p['g%d' % li][None, :], p['be%d' % li][None, :],
                         batch.reshape(N, 1), p['fW1'], p['fb1'][None, :],
                         p['fW2'], p['fb2'][None, :])
    return out


# speedup vs baseline: 1.7281x; 1.1788x over previous
import functools

import jax
import jax.numpy as jnp
from jax import lax
from jax.experimental import pallas as pl
from jax.experimental.pallas import tpu as pltpu
from jax.experimental.pallas import tpu_sc as plsc

N = 10000
E = 320000
H = 128
HH = H // 2
G = 64
EPS = 1e-5

NC = 2
NS = 16
NW = NC * NS
KE = 128
NCH = 79
EPT = NCH * KE
NPAD = 10240
ZCH = NPAD // NS
NWB = ZCH // KE

RB = 1000
NBLK = N // RB

_mesh = plsc.VectorSubcoreMesh(core_axis_name="c", subcore_axis_name="s")


@functools.partial(
    pl.kernel,
    mesh=_mesh,
    out_type=jax.ShapeDtypeStruct((NC, NPAD), jnp.float32),
    scratch_types=[
        pltpu.VMEM_SHARED((NPAD,), jnp.float32),
        pltpu.VMEM((NCH, KE), jnp.int32),
        pltpu.VMEM((NCH, KE), jnp.float32),
        pltpu.VMEM((ZCH,), jnp.float32),
    ],
)
def _deg_call(col_hbm, ew_hbm, out_hbm, acc, colv, ewv, zbuf):
    c = lax.axis_index("c")
    s = lax.axis_index("s")
    wid = c * NS + s
    zero = jnp.zeros((16,), jnp.float32)

    def zb(i, _):
        zbuf[pl.ds(pl.multiple_of(i * 16, 16), 16)] = zero
        return 0

    lax.fori_loop(0, ZCH // 16, zb, 0)
    pltpu.sync_copy(zbuf, acc.at[pl.ds(s * ZCH, ZCH)])
    pltpu.sync_copy(col_hbm.at[wid], colv)
    pltpu.sync_copy(ew_hbm.at[wid], ewv)
    plsc.subcore_barrier()

    def ch(j, _):
        pltpu.sync_copy(ewv.at[j], acc.at[colv.at[j]], add=True)
        return 0

    lax.fori_loop(0, NCH, ch, 0)
    plsc.subcore_barrier()
    pltpu.sync_copy(acc.at[pl.ds(s * ZCH, ZCH)], zbuf)
    pltpu.sync_copy(zbuf, out_hbm.at[c, pl.ds(s * ZCH, ZCH)])


@functools.partial(
    pl.kernel,
    mesh=_mesh,
    out_type=jax.ShapeDtypeStruct((NC, NPAD, H), jnp.float32),
    scratch_types=[
        pltpu.VMEM_SHARED((NPAD, H), jnp.float32),
        pltpu.VMEM((NCH, KE), jnp.int32),
        pltpu.VMEM((NCH, KE), jnp.int32),
        pltpu.VMEM((NCH, KE), jnp.float32),
        pltpu.VMEM((KE, H), jnp.float32),
        pltpu.SemaphoreType.DMA,
    ],
)
def _agg_call(s_hbm, row_hbm, col_hbm, ew_hbm, out_hbm, acc, rowv, colv, ewv,
              buf, gsem):
    c = lax.axis_index("c")
    s = lax.axis_index("s")
    wid = c * NS + s
    zero = jnp.zeros((16,), jnp.float32)

    def zr(r, _):
        for q in range(H // 16):
            buf[r, pl.ds(q * 16, 16)] = zero
        return 0

    lax.fori_loop(0, KE, zr, 0)
    for i in range(NWB):
        pltpu.sync_copy(buf, acc.at[pl.ds(s * ZCH + i * KE, KE)])
    pltpu.sync_copy(row_hbm.at[wid], rowv)
    pltpu.sync_copy(col_hbm.at[wid], colv)
    pltpu.sync_copy(ew_hbm.at[wid], ewv)
    plsc.subcore_barrier()

    def ch(j, _):
        pltpu.async_copy(s_hbm.at[rowv.at[j]], buf, gsem).wait()

        def eb(q2, _):
            base = pl.multiple_of(q2 * 16, 16)
            wv = ewv[j, pl.ds(base, 16)]
            for l in range(16):
                w = wv[l]
                e = base + l
                for q in range(H // 16):
                    buf[e, pl.ds(q * 16, 16)] = buf[e, pl.ds(q * 16, 16)] * w
            return 0

        lax.fori_loop(0, KE // 16, eb, 0)
        pltpu.sync_copy(buf, acc.at[colv.at[j]], add=True)
        return 0

    lax.fori_loop(0, NCH, ch, 0)
    plsc.subcore_barrier()
    for i in range(NWB):
        pltpu.sync_copy(acc.at[pl.ds(s * ZCH + i * KE, KE)], buf)
        pltpu.sync_copy(buf, out_hbm.at[c, pl.ds(s * ZCH + i * KE, KE)])


def _prep_body(deg_ref, x_ref, w_ref, s_ref, dinv_ref):
    deg = deg_ref[0, :, 0] + deg_ref[1, :, 0] + 1.0
    dinv = jnp.where(deg > 0, lax.rsqrt(deg), 0.0)
    dinv_ref[...] = dinv[:, None]
    s_ref[...] = jnp.dot(x_ref[...], w_ref[...],
                         preferred_element_type=jnp.float32) * dinv[:, None]


def _stats_body(a_ref, s_ref, dinv_ref, b_ref, t_ref, st_ref, *, relu_first):
    i = pl.program_id(0)
    o = (a_ref[0] + a_ref[1] + s_ref[...]) * dinv_ref[...] + b_ref[...]
    if relu_first:
        o = jnp.maximum(o, 0.0)
    t_ref[...] = o

    @pl.when(i == 0)
    def _():
        st_ref[...] = jnp.zeros_like(st_ref)

    st_ref[0:1] += jnp.sum(o, axis=0, keepdims=True)
    st_ref[1:2] += jnp.sum(o * o, axis=0, keepdims=True)


def _apply_body(t_ref, st_ref, g_ref, be_ref, w_ref, dinv_ref, o_ref, *,
                relu_after):
    mu = st_ref[0:1] * (1.0 / N)
    var = st_ref[1:2] * (1.0 / N) - mu * mu
    tn = g_ref[...] * (t_ref[...] - mu) / jnp.sqrt(var + EPS) + be_ref[...]
    if relu_after:
        tn = jnp.maximum(tn, 0.0)
    o_ref[...] = jnp.dot(tn, w_ref[...],
                         preferred_element_type=jnp.float32) * dinv_ref[...]


def _final_body(t_ref, st_ref, g_ref, be_ref, batch_ref, fw1_ref, fb1_ref,
                fw2_ref, fb2_ref, o_ref, acc_ref):
    i = pl.program_id(0)
    mu = st_ref[0:1] * (1.0 / N)
    var = st_ref[1:2] * (1.0 / N) - mu * mu
    hn = g_ref[...] * (t_ref[...] - mu) / jnp.sqrt(var + EPS) + be_ref[...]
    gid = lax.broadcasted_iota(jnp.int32, (G, RB), 0)
    onehot = (batch_ref[...][:, 0][None, :] == gid).astype(jnp.float32)

    @pl.when(i == 0)
    def _():
        acc_ref[...] = jnp.zeros_like(acc_ref)

    acc_ref[...] += jnp.dot(onehot, hn, preferred_element_type=jnp.float32)

    @pl.when(i == NBLK - 1)
    def _():
        pooled = jnp.maximum(acc_ref[...], 0.0)
        z = jnp.maximum(
            jnp.dot(pooled, fw1_ref[...], preferred_element_type=jnp.float32)
            + fb1_ref[...], 0.0)
        o_ref[...] = jnp.dot(z, fw2_ref[...],
                             preferred_element_type=jnp.float32) + fb2_ref[...]


_rowspec = pl.BlockSpec((RB, H), lambda i: (i, 0))
_aggspec = pl.BlockSpec((NC, RB, H), lambda i: (0, i, 0))
_vecspec = pl.BlockSpec((RB, 1), lambda i: (i, 0))
_wspec = pl.BlockSpec((H, H), lambda i: (0, 0))
_bspec = pl.BlockSpec((1, H), lambda i: (0, 0))
_stspec = pl.BlockSpec((2, H), lambda i: (0, 0))

_prep = pl.pallas_call(
    _prep_body,
    grid=(NBLK,),
    in_specs=[pl.BlockSpec((NC, RB, 1), lambda i: (0, i, 0)), _rowspec, _wspec],
    out_specs=[_rowspec, _vecspec],
    out_shape=[jax.ShapeDtypeStruct((N, H), jnp.float32),
               jax.ShapeDtypeStruct((N, 1), jnp.float32)],
)


def _stats(relu_first):
    return pl.pallas_call(
        functools.partial(_stats_body, relu_first=relu_first),
        grid=(NBLK,),
        in_specs=[_aggspec, _rowspec, _vecspec, _bspec],
        out_specs=[_rowspec, _stspec],
        out_shape=[jax.ShapeDtypeStruct((N, H), jnp.float32),
                   jax.ShapeDtypeStruct((2, H), jnp.float32)],
    )


def _apply(relu_after):
    return pl.pallas_call(
        functools.partial(_apply_body, relu_after=relu_after),
        grid=(NBLK,),
        in_specs=[_rowspec, _stspec, _bspec, _bspec, _wspec, _vecspec],
        out_specs=_rowspec,
        out_shape=jax.ShapeDtypeStruct((N, H), jnp.float32),
    )


_final = pl.pallas_call(
    _final_body,
    grid=(NBLK,),
    in_specs=[_rowspec, _stspec, _bspec, _bspec,
              pl.BlockSpec((RB, 1), lambda i: (i, 0)), _wspec,
              _bspec, pl.BlockSpec((H, 20), lambda i: (0, 0)),
              pl.BlockSpec((1, 20), lambda i: (0, 0))],
    out_specs=pl.BlockSpec((G, 20), lambda i: (0, 0)),
    out_shape=jax.ShapeDtypeStruct((G, 20), jnp.float32),
    scratch_shapes=[pltpu.VMEM((G, H), jnp.float32)],
)

_ORDER = [(True, False), (True, False), (True, False), (False, True),
          (False, False)]


def kernel(x, edge_index, edge_weight, batch, params):
    p = params
    ept = E // NW
    padw = ((0, 0), (0, EPT - ept))
    row_p = jnp.pad(edge_index[0].reshape(NW, ept), padw).reshape(NW, NCH, KE)
    col_p = jnp.pad(edge_index[1].reshape(NW, ept), padw,
                    constant_values=NPAD - 1).reshape(NW, NCH, KE)
    ew_p = jnp.pad(edge_weight.reshape(NW, ept), padw).reshape(NW, NCH, KE)

    degp = _deg_call(col_p, ew_p)
    degp3 = degp.reshape(NC, NPAD, 1)
    s, dinv = _prep(degp3, x, p['W1'])

    out = None
    for li in range(1, 6):
        relu_first, relu_after = _ORDER[li - 1]
        aggp = _agg_call(s, row_p, col_p, ew_p)
        t, st = _stats(relu_first)(aggp, s, dinv, p['b%d' % li][None, :])
        if li < 5:
            s = _apply(relu_after)(t, st, p['g%d' % li][None, :],
                                   p['be%d' % li][None, :], p['W%d' % (li + 1)],
                                   dinv)
        else:
            out = _final(t, st, p['g%d' % li][None, :], p['be%d' % li][None, :],
                         batch.reshape(N, 1), p['fW1'], p['fb1'][None, :],
                         p['fW2'], p['fb2'][None, :])
    return out
